# baseline (speedup 1.0000x reference)
import functools

import jax
import jax.numpy as jnp
from jax import lax
from jax.experimental import pallas as pl
from jax.experimental.pallas import tpu as pltpu
from jax.experimental.pallas import tpu_sc as plsc

_TAU = 1.0
_LANES = 16
_SUBCORES = 16
_PAD_G = 1e30


def _tc_body(u_ref, it_ref, wq_ref, bq_ref, wk_ref, bk_ref,
             rows_ref, cols_ref, n1_ref, n2_ref,
             w1_ref, w2_ref, flat_ref, g1_ref, g2_ref,
             qu_s, ku_s, qi_s, ki_s, *, nnz):
    j = pl.program_id(0)

    @pl.when(j == 0)
    def _():
        u = u_ref[...]
        it = it_ref[...]
        wq = wq_ref[...]
        wk = wk_ref[...]
        bq = bq_ref[...]
        bk = bk_ref[...]
        qu_s[...] = jnp.dot(u, wq, preferred_element_type=jnp.float32) + bq
        ku_s[...] = jnp.dot(u, wk, preferred_element_type=jnp.float32) + bk
        qi_s[...] = jnp.dot(it, wq, preferred_element_type=jnp.float32) + bq
        ki_s[...] = jnp.dot(it, wk, preferred_element_type=jnp.float32) + bk

        rows = rows_ref[...]
        cols = cols_ref[...]
        n_users = u.shape[0]
        jb = cols // 128
        flat_ref[...] = jb * (n_users * 128) + rows * 128 + (cols - jb * 128)
        shp = rows.shape
        pos = (lax.broadcasted_iota(jnp.int32, shp, 0) * shp[1]
               + lax.broadcasted_iota(jnp.int32, shp, 1))
        pad = (pos >= nnz).astype(jnp.float32) * _PAD_G
        g1_ref[...] = jnp.log(-jnp.log(n1_ref[...])) + pad
        g2_ref[...] = jnp.log(-jnp.log(n2_ref[...])) + pad

    dn = (((1,), (1,)), ((), ()))
    for t in range(2):
        ki_blk = ki_s[pl.ds((2 * j + t) * 128, 128), :]
        qi_blk = qi_s[pl.ds((2 * j + t) * 128, 128), :]
        w1_ref[t] = lax.dot_general(qu_s[...], ki_blk, dn,
                                    preferred_element_type=jnp.float32)
        w2_ref[t] = lax.dot_general(ku_s[...], qi_blk, dn,
                                    preferred_element_type=jnp.float32)


def _sc_body(w1_hbm, w2_hbm, flat_hbm, segu_hbm, segi_hbm, g1_hbm, g2_hbm,
             out1_hbm, out2_hbm,
             idx_v, seg_v, gg_v, vals_v, ex_v, ssum_v, ssum_sh, sem,
             *, chunk, nseg):
    c = lax.axis_index("c")
    s = lax.axis_index("s")
    base = s * chunk
    nvec = chunk // _LANES

    @pl.when(s == 0)
    def _():
        def zbody(i, carry):
            ssum_v[pl.ds(i * _LANES, _LANES)] = jnp.zeros((_LANES,), jnp.float32)
            return carry
        lax.fori_loop(0, nseg // _LANES, zbody, 0)
        pltpu.sync_copy(ssum_v, ssum_sh)

    pltpu.sync_copy(flat_hbm.at[pl.ds(base, chunk)], idx_v)

    @pl.when(c == 0)
    def _():
        cp = pltpu.async_copy(w1_hbm.at[idx_v], vals_v, sem)
        pltpu.sync_copy(segu_hbm.at[pl.ds(base, chunk)], seg_v)
        pltpu.sync_copy(g1_hbm.at[pl.ds(base, chunk)], gg_v)
        cp.wait()

    @pl.when(c != 0)
    def _():
        cp = pltpu.async_copy(w2_hbm.at[idx_v], vals_v, sem)
        pltpu.sync_copy(segi_hbm.at[pl.ds(base, chunk)], seg_v)
        pltpu.sync_copy(g2_hbm.at[pl.ds(base, chunk)], gg_v)
        cp.wait()

    def ebody(i, carry):
        sl = pl.ds(i * _LANES, _LANES)
        ex_v[sl] = jnp.exp((vals_v[sl] - gg_v[sl]) * (1.0 / _TAU))
        return carry
    lax.fori_loop(0, nvec, ebody, 0)

    plsc.subcore_barrier()

    pltpu.sync_copy(ex_v, ssum_sh.at[seg_v], add=True)
    plsc.subcore_barrier()

    @pl.when(s == 0)
    def _():
        pltpu.sync_copy(ssum_sh, ssum_v)

        def ibody(i, carry):
            sl = pl.ds(i * _LANES, _LANES)
            ssum_v[sl] = 1.0 / ssum_v[sl]
            return carry
        lax.fori_loop(0, nseg // _LANES, ibody, 0)
        pltpu.sync_copy(ssum_v, ssum_sh)

    plsc.subcore_barrier()

    pltpu.async_copy(ssum_sh.at[seg_v], gg_v, sem).wait()

    def nbody(i, carry):
        sl = pl.ds(i * _LANES, _LANES)
        vals_v[sl] = ex_v[sl] * gg_v[sl]
        return carry
    lax.fori_loop(0, nvec, nbody, 0)

    @pl.when(c == 0)
    def _():
        pltpu.sync_copy(vals_v, out1_hbm.at[pl.ds(base, chunk)])

    @pl.when(c != 0)
    def _():
        pltpu.sync_copy(vals_v, out2_hbm.at[pl.ds(base, chunk)])


def kernel(user_embed, item_embed, ui_indices, noise_ui, noise_iu, Wq, bq, Wk, bk):
    n_users, embed = user_embed.shape
    n_items = item_embed.shape[0]
    att = Wq.shape[1]
    nnz = noise_ui.shape[0]

    lcm = _SUBCORES * 128
    nnz_pad = ((nnz + lcm - 1) // lcm) * lcm
    chunk = nnz_pad // _SUBCORES
    pad = nnz_pad - nnz

    rows = ui_indices[0].astype(jnp.int32)
    cols = ui_indices[1].astype(jnp.int32)
    rows_p = jnp.pad(rows, (0, pad)).reshape(nnz_pad // 128, 128)
    cols_p = jnp.pad(cols, (0, pad)).reshape(nnz_pad // 128, 128)
    n1_p = jnp.pad(noise_ui, (0, pad), constant_values=0.5).reshape(nnz_pad // 128, 128)
    n2_p = jnp.pad(noise_iu, (0, pad), constant_values=0.5).reshape(nnz_pad // 128, 128)
    bq2 = bq.reshape(1, att)
    bk2 = bk.reshape(1, att)

    jblk = n_items // 128
    nrow = nnz_pad // 128
    tc = pl.pallas_call(
        functools.partial(_tc_body, nnz=nnz),
        grid=(jblk // 2,),
        in_specs=[
            pl.BlockSpec((n_users, embed), lambda j: (0, 0)),
            pl.BlockSpec((n_items, embed), lambda j: (0, 0)),
            pl.BlockSpec((embed, att), lambda j: (0, 0)),
            pl.BlockSpec((1, att), lambda j: (0, 0)),
            pl.BlockSpec((embed, att), lambda j: (0, 0)),
            pl.BlockSpec((1, att), lambda j: (0, 0)),
            pl.BlockSpec((nrow, 128), lambda j: (0, 0)),
            pl.BlockSpec((nrow, 128), lambda j: (0, 0)),
            pl.BlockSpec((nrow, 128), lambda j: (0, 0)),
            pl.BlockSpec((nrow, 128), lambda j: (0, 0)),
        ],
        out_specs=[
            pl.BlockSpec((2, n_users, 128), lambda j: (j, 0, 0)),
            pl.BlockSpec((2, n_users, 128), lambda j: (j, 0, 0)),
            pl.BlockSpec((nrow, 128), lambda j: (0, 0)),
            pl.BlockSpec((nrow, 128), lambda j: (0, 0)),
            pl.BlockSpec((nrow, 128), lambda j: (0, 0)),
        ],
        out_shape=[
            jax.ShapeDtypeStruct((jblk, n_users, 128), jnp.float32),
            jax.ShapeDtypeStruct((jblk, n_users, 128), jnp.float32),
            jax.ShapeDtypeStruct((nrow, 128), jnp.int32),
            jax.ShapeDtypeStruct((nrow, 128), jnp.float32),
            jax.ShapeDtypeStruct((nrow, 128), jnp.float32),
        ],
        scratch_shapes=[
            pltpu.VMEM((n_users, att), jnp.float32),
            pltpu.VMEM((n_users, att), jnp.float32),
            pltpu.VMEM((n_items, att), jnp.float32),
            pltpu.VMEM((n_items, att), jnp.float32),
        ],
    )
    w1, w2, flat, g1, g2 = tc(
        user_embed, item_embed, Wq, bq2, Wk, bk2, rows_p, cols_p, n1_p, n2_p)
    w1f = w1.reshape(n_users * n_items)
    w2f = w2.reshape(n_users * n_items)

    nseg = n_users
    sc = functools.partial(
        pl.kernel,
        mesh=plsc.VectorSubcoreMesh(core_axis_name="c", subcore_axis_name="s"),
        out_type=[
            jax.ShapeDtypeStruct((nnz_pad,), jnp.float32),
            jax.ShapeDtypeStruct((nnz_pad,), jnp.float32),
        ],
        scratch_types=[
            pltpu.VMEM((chunk,), jnp.int32),
            pltpu.VMEM((chunk,), jnp.int32),
            pltpu.VMEM((chunk,), jnp.float32),
            pltpu.VMEM((chunk,), jnp.float32),
            pltpu.VMEM((chunk,), jnp.float32),
            pltpu.VMEM((nseg,), jnp.float32),
            pltpu.VMEM_SHARED((nseg,), jnp.float32),
            pltpu.SemaphoreType.DMA,
        ],
    )(functools.partial(_sc_body, chunk=chunk, nseg=nseg))
    out1, out2 = sc(w1f, w2f, flat.reshape(nnz_pad),
                    rows_p.reshape(nnz_pad), cols_p.reshape(nnz_pad),
                    g1.reshape(nnz_pad), g2.reshape(nnz_pad))
    return jnp.concatenate([out1[:nnz], out2[:nnz]], axis=0)


# speedup vs baseline: 1.0320x; 1.0320x over previous
import functools

import jax
import jax.numpy as jnp
from jax import lax
from jax.experimental import pallas as pl
from jax.experimental.pallas import tpu as pltpu
from jax.experimental.pallas import tpu_sc as plsc

_TAU = 1.0
_LANES = 16
_SUBCORES = 16
_PAD_G = 1e30


def _tc_body(u_ref, it_ref, wq_ref, bq_ref, wk_ref, bk_ref,
             rows_ref, cols_ref, n1_ref, n2_ref,
             w1_ref, w2_ref, flat_ref, g1_ref, g2_ref,
             qu_s, ku_s, qi_s, ki_s, *, nnz):
    j = pl.program_id(0)

    @pl.when(j == 0)
    def _():
        u = u_ref[...]
        it = it_ref[...]
        wq = wq_ref[...]
        wk = wk_ref[...]
        bq = bq_ref[...]
        bk = bk_ref[...]
        qu_s[...] = jnp.dot(u, wq, preferred_element_type=jnp.float32) + bq
        ku_s[...] = jnp.dot(u, wk, preferred_element_type=jnp.float32) + bk
        qi_s[...] = jnp.dot(it, wq, preferred_element_type=jnp.float32) + bq
        ki_s[...] = jnp.dot(it, wk, preferred_element_type=jnp.float32) + bk

        rows = rows_ref[...]
        cols = cols_ref[...]
        n_users = u.shape[0]
        jb = cols // 128
        flat_ref[...] = jb * (n_users * 128) + rows * 128 + (cols - jb * 128)
        shp = rows.shape
        pos = (lax.broadcasted_iota(jnp.int32, shp, 0) * shp[1]
               + lax.broadcasted_iota(jnp.int32, shp, 1))
        pad = (pos >= nnz).astype(jnp.float32) * _PAD_G
        g1_ref[...] = jnp.log(-jnp.log(n1_ref[...])) + pad
        g2_ref[...] = jnp.log(-jnp.log(n2_ref[...])) + pad

    dn = (((1,), (1,)), ((), ()))
    for t in range(2):
        ki_blk = ki_s[pl.ds((2 * j + t) * 128, 128), :]
        qi_blk = qi_s[pl.ds((2 * j + t) * 128, 128), :]
        w1_ref[t] = lax.dot_general(qu_s[...], ki_blk, dn,
                                    preferred_element_type=jnp.float32)
        w2_ref[t] = lax.dot_general(ku_s[...], qi_blk, dn,
                                    preferred_element_type=jnp.float32)


def _sc_body(w1_hbm, w2_hbm, flat_hbm, segu_hbm, segi_hbm, g1_hbm, g2_hbm,
             out1_hbm, out2_hbm,
             idx_v, seg_v, gg_v, vals_v, ex_v, ssum_v, ssum_sh, sem,
             *, chunk, nseg):
    c = lax.axis_index("c")
    s = lax.axis_index("s")
    base = s * chunk
    nvec = chunk // _LANES

    @pl.when(s == 0)
    def _():
        def zbody(i, carry):
            ssum_v[pl.ds(i * _LANES, _LANES)] = jnp.zeros((_LANES,), jnp.float32)
            return carry
        lax.fori_loop(0, nseg // _LANES, zbody, 0)
        pltpu.sync_copy(ssum_v, ssum_sh)

    pltpu.sync_copy(flat_hbm.at[pl.ds(base, chunk)], idx_v)

    @pl.when(c == 0)
    def _():
        cp = pltpu.async_copy(w1_hbm.at[idx_v], vals_v, sem)
        pltpu.sync_copy(segu_hbm.at[pl.ds(base, chunk)], seg_v)
        pltpu.sync_copy(g1_hbm.at[pl.ds(base, chunk)], gg_v)
        cp.wait()

    @pl.when(c != 0)
    def _():
        cp = pltpu.async_copy(w2_hbm.at[idx_v], vals_v, sem)
        pltpu.sync_copy(segi_hbm.at[pl.ds(base, chunk)], seg_v)
        pltpu.sync_copy(g2_hbm.at[pl.ds(base, chunk)], gg_v)
        cp.wait()

    def ebody(i, carry):
        sl = pl.ds(i * _LANES, _LANES)
        ex_v[sl] = jnp.exp((vals_v[sl] - gg_v[sl]) * (1.0 / _TAU))
        return carry
    lax.fori_loop(0, nvec, ebody, 0)

    plsc.subcore_barrier()

    pltpu.sync_copy(ex_v, ssum_sh.at[seg_v], add=True)
    plsc.subcore_barrier()

    pltpu.async_copy(ssum_sh.at[seg_v], gg_v, sem).wait()

    def nbody(i, carry):
        sl = pl.ds(i * _LANES, _LANES)
        vals_v[sl] = ex_v[sl] / gg_v[sl]
        return carry
    lax.fori_loop(0, nvec, nbody, 0)

    @pl.when(c == 0)
    def _():
        pltpu.sync_copy(vals_v, out1_hbm.at[pl.ds(base, chunk)])

    @pl.when(c != 0)
    def _():
        pltpu.sync_copy(vals_v, out2_hbm.at[pl.ds(base, chunk)])


def kernel(user_embed, item_embed, ui_indices, noise_ui, noise_iu, Wq, bq, Wk, bk):
    n_users, embed = user_embed.shape
    n_items = item_embed.shape[0]
    att = Wq.shape[1]
    nnz = noise_ui.shape[0]

    lcm = _SUBCORES * 128
    nnz_pad = ((nnz + lcm - 1) // lcm) * lcm
    chunk = nnz_pad // _SUBCORES
    pad = nnz_pad - nnz

    rows = ui_indices[0].astype(jnp.int32)
    cols = ui_indices[1].astype(jnp.int32)
    rows_p = jnp.pad(rows, (0, pad)).reshape(nnz_pad // 128, 128)
    cols_p = jnp.pad(cols, (0, pad)).reshape(nnz_pad // 128, 128)
    n1_p = jnp.pad(noise_ui, (0, pad), constant_values=0.5).reshape(nnz_pad // 128, 128)
    n2_p = jnp.pad(noise_iu, (0, pad), constant_values=0.5).reshape(nnz_pad // 128, 128)
    bq2 = bq.reshape(1, att)
    bk2 = bk.reshape(1, att)

    jblk = n_items // 128
    nrow = nnz_pad // 128
    tc = pl.pallas_call(
        functools.partial(_tc_body, nnz=nnz),
        grid=(jblk // 2,),
        in_specs=[
            pl.BlockSpec((n_users, embed), lambda j: (0, 0)),
            pl.BlockSpec((n_items, embed), lambda j: (0, 0)),
            pl.BlockSpec((embed, att), lambda j: (0, 0)),
            pl.BlockSpec((1, att), lambda j: (0, 0)),
            pl.BlockSpec((embed, att), lambda j: (0, 0)),
            pl.BlockSpec((1, att), lambda j: (0, 0)),
            pl.BlockSpec((nrow, 128), lambda j: (0, 0)),
            pl.BlockSpec((nrow, 128), lambda j: (0, 0)),
            pl.BlockSpec((nrow, 128), lambda j: (0, 0)),
            pl.BlockSpec((nrow, 128), lambda j: (0, 0)),
        ],
        out_specs=[
            pl.BlockSpec((2, n_users, 128), lambda j: (j, 0, 0)),
            pl.BlockSpec((2, n_users, 128), lambda j: (j, 0, 0)),
            pl.BlockSpec((nrow, 128), lambda j: (0, 0)),
            pl.BlockSpec((nrow, 128), lambda j: (0, 0)),
            pl.BlockSpec((nrow, 128), lambda j: (0, 0)),
        ],
        out_shape=[
            jax.ShapeDtypeStruct((jblk, n_users, 128), jnp.float32),
            jax.ShapeDtypeStruct((jblk, n_users, 128), jnp.float32),
            jax.ShapeDtypeStruct((nrow, 128), jnp.int32),
            jax.ShapeDtypeStruct((nrow, 128), jnp.float32),
            jax.ShapeDtypeStruct((nrow, 128), jnp.float32),
        ],
        scratch_shapes=[
            pltpu.VMEM((n_users, att), jnp.float32),
            pltpu.VMEM((n_users, att), jnp.float32),
            pltpu.VMEM((n_items, att), jnp.float32),
            pltpu.VMEM((n_items, att), jnp.float32),
        ],
    )
    w1, w2, flat, g1, g2 = tc(
        user_embed, item_embed, Wq, bq2, Wk, bk2, rows_p, cols_p, n1_p, n2_p)
    w1f = w1.reshape(n_users * n_items)
    w2f = w2.reshape(n_users * n_items)

    nseg = n_users
    sc = functools.partial(
        pl.kernel,
        mesh=plsc.VectorSubcoreMesh(core_axis_name="c", subcore_axis_name="s"),
        out_type=[
            jax.ShapeDtypeStruct((nnz_pad,), jnp.float32),
            jax.ShapeDtypeStruct((nnz_pad,), jnp.float32),
        ],
        scratch_types=[
            pltpu.VMEM((chunk,), jnp.int32),
            pltpu.VMEM((chunk,), jnp.int32),
            pltpu.VMEM((chunk,), jnp.float32),
            pltpu.VMEM((chunk,), jnp.float32),
            pltpu.VMEM((chunk,), jnp.float32),
            pltpu.VMEM((nseg,), jnp.float32),
            pltpu.VMEM_SHARED((nseg,), jnp.float32),
            pltpu.SemaphoreType.DMA,
        ],
    )(functools.partial(_sc_body, chunk=chunk, nseg=nseg))
    out1, out2 = sc(w1f, w2f, flat.reshape(nnz_pad),
                    rows_p.reshape(nnz_pad), cols_p.reshape(nnz_pad),
                    g1.reshape(nnz_pad), g2.reshape(nnz_pad))
    return jnp.concatenate([out1[:nnz], out2[:nnz]], axis=0)
